# baseline (speedup 1.0000x reference)
import functools

import jax
import jax.numpy as jnp
from jax.experimental import pallas as pl
from jax.experimental.pallas import tpu as pltpu

N_NODES = 10000
NODE_DIM = 256
HID = 512
BLK = 2000
GRID = N_NODES // BLK
EPS = 1e-5


def _mlp_body(h0_ref, w1_ref, b1_ref, w2_ref, b2_ref, h2_ref, part_ref):
    h0 = h0_ref[...]
    h1 = jnp.dot(h0, w1_ref[...], preferred_element_type=jnp.float32) + b1_ref[...]
    h1 = jnp.maximum(h1, 0.0)
    h2 = jnp.dot(h1, w2_ref[...], preferred_element_type=jnp.float32) + b2_ref[...]
    h2_ref[...] = h2
    part_ref[0, 0, :] = jnp.sum(h2, axis=0)
    part_ref[0, 1, :] = jnp.sum(h2 * h2, axis=0)


def _mlp_call(h0, w1, b1, w2, b2):
    return pl.pallas_call(
        _mlp_body,
        grid=(GRID,),
        in_specs=[
            pl.BlockSpec((BLK, NODE_DIM), lambda i: (i, 0)),
            pl.BlockSpec((NODE_DIM, HID), lambda i: (0, 0)),
            pl.BlockSpec((1, HID), lambda i: (0, 0)),
            pl.BlockSpec((HID, NODE_DIM), lambda i: (0, 0)),
            pl.BlockSpec((1, NODE_DIM), lambda i: (0, 0)),
        ],
        out_specs=[
            pl.BlockSpec((BLK, NODE_DIM), lambda i: (i, 0)),
            pl.BlockSpec((1, 2, NODE_DIM), lambda i: (i, 0, 0)),
        ],
        out_shape=[
            jax.ShapeDtypeStruct((N_NODES, NODE_DIM), jnp.float32),
            jax.ShapeDtypeStruct((GRID, 2, NODE_DIM), jnp.float32),
        ],
    )(h0, w1, b1.reshape(1, HID), w2, b2.reshape(1, NODE_DIM))


def _bn_body(h2_ref, part_ref, res_ref, gamma_ref, beta_ref, out_ref):
    s = jnp.sum(part_ref[:, 0, :], axis=0)
    ss = jnp.sum(part_ref[:, 1, :], axis=0)
    mean = s / N_NODES
    var = ss / N_NODES - mean * mean
    rstd = jax.lax.rsqrt(var + EPS)
    h = (h2_ref[...] - mean) * (rstd * gamma_ref[0]) + (beta_ref[0] - mean * 0.0)
    h = jnp.maximum(h, 0.0)
    out_ref[...] = h + res_ref[...]


def _bn_call(h2, part, res, gamma, beta):
    return pl.pallas_call(
        _bn_body,
        grid=(GRID,),
        in_specs=[
            pl.BlockSpec((BLK, NODE_DIM), lambda i: (i, 0)),
            pl.BlockSpec((GRID, 2, NODE_DIM), lambda i: (0, 0, 0)),
            pl.BlockSpec((BLK, NODE_DIM), lambda i: (i, 0)),
            pl.BlockSpec((1, NODE_DIM), lambda i: (0, 0)),
            pl.BlockSpec((1, NODE_DIM), lambda i: (0, 0)),
        ],
        out_specs=pl.BlockSpec((BLK, NODE_DIM), lambda i: (i, 0)),
        out_shape=jax.ShapeDtypeStruct((N_NODES, NODE_DIM), jnp.float32),
    )(h2, part, res, gamma.reshape(1, NODE_DIM), beta.reshape(1, NODE_DIM))


def kernel(atom_type, edge_index, edge_type, atom_emb, edge_emb, W1, b1, W2, b2, gamma, beta):
    num_layers = W1.shape[0]
    x = jnp.take(atom_emb, atom_type, axis=0)
    e = jnp.take(edge_emb, edge_type, axis=0)
    src = edge_index[0]
    dst = edge_index[1]
    for l in range(num_layers):
        msg = jax.nn.relu(jnp.take(x, src, axis=0) + e)
        aggr = jax.ops.segment_sum(msg, dst, num_segments=N_NODES)
        h0 = aggr + x
        h2, part = _mlp_call(h0, W1[l], b1[l], W2[l], b2[l])
        x = _bn_call(h2, part, x, gamma[l], beta[l])
    return x
